# baseline (speedup 1.0000x reference)
import functools

import jax
import jax.numpy as jnp
from jax import lax
from jax.experimental import pallas as pl
from jax.experimental.pallas import tpu as pltpu
from jax.experimental.pallas import tpu_sc as plsc

B = 16384
TB = 4096
NW = 32
RPW = B // NW
NCH = RPW // 128


NS_, NG_, NM_ = 102, 22, 12
NT2 = NS_ * NG_


def _prep_body(se_ref, ge_ref, me_ref, Wsub_ref, bsub_ref, Wgr_ref, bgr_ref,
               W1_ref, b1_ref, si_ref, gi_ref,
               t2_o, tmb_o, wsub1_o, wgr1_o, ip_o):
    f32 = jnp.float32
    W1 = W1_ref[...]
    ts = jnp.dot(se_ref[...], W1[0:32, :], preferred_element_type=f32)
    tg = jnp.dot(ge_ref[...], W1[32:64, :], preferred_element_type=f32)
    bias = (b1_ref[...]
            + jnp.dot(bsub_ref[...], W1[96:128, :], preferred_element_type=f32)
            + jnp.dot(bgr_ref[...], W1[128:160, :], preferred_element_type=f32))
    tmb_o[...] = jnp.dot(me_ref[...], W1[64:96, :], preferred_element_type=f32) + bias
    t2_o[...] = (ts[:, None, :] + tg[None, :, :]).reshape(NT2, 128)
    wsub1_o[...] = jnp.dot(Wsub_ref[...], W1[96:128, :], preferred_element_type=f32)
    wgr1_o[...] = jnp.dot(Wgr_ref[...], W1[128:160, :], preferred_element_type=f32)
    ip_o[...] = si_ref[...] * NG_ + gi_ref[...]


def _prep(school_emb, goal_emb, method_emb, W_sub, b_sub, W_gr, b_gr, W1, b1,
          si2, gi2):
    return pl.pallas_call(
        _prep_body,
        out_shape=(
            jax.ShapeDtypeStruct((NT2, 128), jnp.float32),
            jax.ShapeDtypeStruct((12, 128), jnp.float32),
            jax.ShapeDtypeStruct((15, 128), jnp.float32),
            jax.ShapeDtypeStruct((12, 128), jnp.float32),
            jax.ShapeDtypeStruct((B // 128, 128), jnp.int32),
        ),
    )(school_emb, goal_emb, method_emb, W_sub, b_sub.reshape(1, 32),
      W_gr, b_gr.reshape(1, 32), W1, b1.reshape(1, 128), si2, gi2)


CH = 128
NSET = 4


def _sc_body(ip_hbm, t3_hbm, out_hbm, idx_p, bufs, gsem, wsem):
    cid = lax.axis_index("c")
    sid = lax.axis_index("s")
    wid = cid * 16 + sid
    gbase = wid * RPW
    grow = wid * NCH

    pltpu.sync_copy(ip_hbm.at[pl.ds(grow, NCH)], idx_p)

    gh = [pltpu.async_copy(t3_hbm.at[idx_p.at[p]], bufs.at[p], gsem.at[p])
          for p in range(NCH)]
    wh = []
    for q in range(NCH):
        gh[q].wait()
        wh.append(pltpu.async_copy(
            bufs.at[q], out_hbm.at[pl.ds(gbase + q * CH, CH)], wsem.at[q]))
    for h in wh:
        h.wait()


def _sc_gather(ip2, t3):
    mesh = plsc.VectorSubcoreMesh(core_axis_name="c", subcore_axis_name="s")
    k = functools.partial(
        pl.kernel,
        mesh=mesh,
        out_type=jax.ShapeDtypeStruct((B, 128), jnp.float32),
        scratch_types=[
            pltpu.VMEM((NCH, 128), jnp.int32),
            pltpu.VMEM((NSET, CH, 128), jnp.float32),
            pltpu.SemaphoreType.DMA((NSET,)),
            pltpu.SemaphoreType.DMA((NSET,)),
        ],
    )(_sc_body)
    return k(ip2, t3)


def _q_body(mi_ref, subM_ref, grM_ref, tmb_ref, wsub1_ref, wgr1_ref, q_ref):
    f32 = jnp.float32
    oh_m = (mi_ref[0, 0, :][:, None]
            == lax.broadcasted_iota(jnp.int32, (TB, 12), 1)).astype(f32)
    q_ref[...] = (jnp.dot(oh_m, tmb_ref[...], preferred_element_type=f32)
                  + jnp.dot(subM_ref[...], wsub1_ref[...], preferred_element_type=f32)
                  + jnp.dot(grM_ref[...], wgr1_ref[...], preferred_element_type=f32))


def _q(mi3, subM, grM, tmb, wsub1, wgr1):
    nb = B // TB

    def batch_spec(w):
        return pl.BlockSpec((TB, w), lambda i: (i, 0))

    def full_spec(shape):
        return pl.BlockSpec(shape, lambda i: (0,) * len(shape))

    return pl.pallas_call(
        _q_body,
        grid=(nb,),
        in_specs=[
            pl.BlockSpec((1, 1, TB), lambda i: (i, 0, 0)),
            batch_spec(15), batch_spec(12),
            full_spec((12, 128)), full_spec((15, 128)), full_spec((12, 128)),
        ],
        out_specs=pl.BlockSpec((TB, 128), lambda i: (i, 0)),
        out_shape=jax.ShapeDtypeStruct((B, 128), jnp.float32),
    )(mi3, subM, grM, tmb, wsub1, wgr1)


def _fin_body(e_ref, q_ref, W2_ref, b2_ref, W3_ref, b3_ref, out_ref):
    f32 = jnp.float32
    h1 = jnp.maximum(e_ref[...] + q_ref[...], 0.0)
    h2 = jnp.maximum(jnp.dot(h1, W2_ref[...], preferred_element_type=f32) + b2_ref[...], 0.0)
    out_ref[...] = jnp.dot(h2, W3_ref[...], preferred_element_type=f32) + b3_ref[...]


def _fin(E, Q, W2, b2, W3, b3):
    nb = B // TB

    def batch_spec(w):
        return pl.BlockSpec((TB, w), lambda i: (i, 0))

    def full_spec(shape):
        return pl.BlockSpec(shape, lambda i: (0,) * len(shape))

    return pl.pallas_call(
        _fin_body,
        grid=(nb,),
        in_specs=[
            batch_spec(128), batch_spec(128),
            full_spec((128, 64)), full_spec((1, 64)),
            full_spec((64, 32)), full_spec((1, 32)),
        ],
        out_specs=pl.BlockSpec((TB, 32), lambda i: (i, 0)),
        out_shape=jax.ShapeDtypeStruct((B, 32), jnp.float32),
    )(E, Q, W2, b2.reshape(1, 64), W3, b3.reshape(1, 32))


def kernel(school_idx, goal_idx, method_idx, subject_multi_hot, grade_multi_hot,
           school_emb, goal_emb, method_emb, W_sub, b_sub, W_gr, b_gr,
           W1, b1, W2, b2, W3, b3):
    si2 = school_idx.astype(jnp.int32).reshape(B // 128, 128)
    gi2 = goal_idx.astype(jnp.int32).reshape(B // 128, 128)
    mi3 = method_idx.astype(jnp.int32).reshape(B // TB, 1, TB)
    t2, tmb, wsub1, wgr1, ip2 = _prep(
        school_emb, goal_emb, method_emb, W_sub, b_sub, W_gr, b_gr, W1, b1,
        si2, gi2)
    E = _sc_gather(ip2, t2)
    Q = _q(mi3, subject_multi_hot, grade_multi_hot, tmb, wsub1, wgr1)
    return _fin(E, Q, W2, b2, W3, b3)


# speedup vs baseline: 1.1149x; 1.1149x over previous
import functools

import jax
import jax.numpy as jnp
from jax import lax
from jax.experimental import pallas as pl
from jax.experimental.pallas import tpu as pltpu
from jax.experimental.pallas import tpu_sc as plsc

B = 16384
TB = 8192
NW = 32
RPW = B // NW
NCH = RPW // 128


NS_, NG_, NM_ = 102, 22, 12
NT2 = NS_ * NG_


def _prep_body(se_ref, ge_ref, me_ref, Wsub_ref, bsub_ref, Wgr_ref, bgr_ref,
               W1_ref, b1_ref, si_ref, gi_ref,
               t2_o, tmb_o, wsub1_o, wgr1_o, ip_o):
    f32 = jnp.float32
    W1 = W1_ref[...]
    ts = jnp.dot(se_ref[...], W1[0:32, :], preferred_element_type=f32)
    tg = jnp.dot(ge_ref[...], W1[32:64, :], preferred_element_type=f32)
    bias = (b1_ref[...]
            + jnp.dot(bsub_ref[...], W1[96:128, :], preferred_element_type=f32)
            + jnp.dot(bgr_ref[...], W1[128:160, :], preferred_element_type=f32))
    tmb_o[...] = jnp.dot(me_ref[...], W1[64:96, :], preferred_element_type=f32) + bias
    t2_o[...] = (ts[:, None, :] + tg[None, :, :]).reshape(NT2, 128)
    wsub1_o[...] = jnp.dot(Wsub_ref[...], W1[96:128, :], preferred_element_type=f32)
    wgr1_o[...] = jnp.dot(Wgr_ref[...], W1[128:160, :], preferred_element_type=f32)
    ip_o[...] = si_ref[...] * NG_ + gi_ref[...]


def _prep(school_emb, goal_emb, method_emb, W_sub, b_sub, W_gr, b_gr, W1, b1,
          si2, gi2):
    return pl.pallas_call(
        _prep_body,
        out_shape=(
            jax.ShapeDtypeStruct((NT2, 128), jnp.float32),
            jax.ShapeDtypeStruct((12, 128), jnp.float32),
            jax.ShapeDtypeStruct((15, 128), jnp.float32),
            jax.ShapeDtypeStruct((12, 128), jnp.float32),
            jax.ShapeDtypeStruct((B // 128, 128), jnp.int32),
        ),
    )(school_emb, goal_emb, method_emb, W_sub, b_sub.reshape(1, 32),
      W_gr, b_gr.reshape(1, 32), W1, b1.reshape(1, 128), si2, gi2)


CH = 128
NSET = 4


def _sc_body(ip_hbm, t3_hbm, out_hbm, idx_p, bufs, gsem, wsem):
    cid = lax.axis_index("c")
    sid = lax.axis_index("s")
    wid = cid * 16 + sid
    gbase = wid * RPW
    grow = wid * NCH

    pltpu.sync_copy(ip_hbm.at[pl.ds(grow, NCH)], idx_p)

    gh = [pltpu.async_copy(t3_hbm.at[idx_p.at[p]], bufs.at[p], gsem.at[p])
          for p in range(NCH)]
    wh = []
    for q in range(NCH):
        gh[q].wait()
        wh.append(pltpu.async_copy(
            bufs.at[q], out_hbm.at[pl.ds(gbase + q * CH, CH)], wsem.at[q]))
    for h in wh:
        h.wait()


def _sc_gather(ip2, t3):
    mesh = plsc.VectorSubcoreMesh(core_axis_name="c", subcore_axis_name="s")
    k = functools.partial(
        pl.kernel,
        mesh=mesh,
        out_type=jax.ShapeDtypeStruct((B, 128), jnp.float32),
        scratch_types=[
            pltpu.VMEM((NCH, 128), jnp.int32),
            pltpu.VMEM((NSET, CH, 128), jnp.float32),
            pltpu.SemaphoreType.DMA((NSET,)),
            pltpu.SemaphoreType.DMA((NSET,)),
        ],
    )(_sc_body)
    return k(ip2, t3)


def _tail_body(e_ref, mi_ref, subM_ref, grM_ref, tmb_ref, wsub1_ref, wgr1_ref,
               W2_ref, b2_ref, W3_ref, b3_ref, out_ref):
    f32 = jnp.float32
    oh_m = (mi_ref[0, 0, :][:, None]
            == lax.broadcasted_iota(jnp.int32, (TB, 12), 1)).astype(f32)
    h1 = (e_ref[...]
          + jnp.dot(oh_m, tmb_ref[...], preferred_element_type=f32)
          + jnp.dot(subM_ref[...], wsub1_ref[...], preferred_element_type=f32)
          + jnp.dot(grM_ref[...], wgr1_ref[...], preferred_element_type=f32))
    h1 = jnp.maximum(h1, 0.0)
    h2 = jnp.maximum(jnp.dot(h1, W2_ref[...], preferred_element_type=f32) + b2_ref[...], 0.0)
    out_ref[...] = jnp.dot(h2, W3_ref[...], preferred_element_type=f32) + b3_ref[...]


def _tail(E, mi3, subM, grM, tmb, wsub1, wgr1, W2, b2, W3, b3):
    nb = B // TB

    def batch_spec(w):
        return pl.BlockSpec((TB, w), lambda i: (i, 0))

    def full_spec(shape):
        return pl.BlockSpec(shape, lambda i: (0,) * len(shape))

    return pl.pallas_call(
        _tail_body,
        grid=(nb,),
        in_specs=[
            batch_spec(128), pl.BlockSpec((1, 1, TB), lambda i: (i, 0, 0)),
            batch_spec(15), batch_spec(12),
            full_spec((12, 128)), full_spec((15, 128)), full_spec((12, 128)),
            full_spec((128, 64)), full_spec((1, 64)),
            full_spec((64, 32)), full_spec((1, 32)),
        ],
        out_specs=pl.BlockSpec((TB, 32), lambda i: (i, 0)),
        out_shape=jax.ShapeDtypeStruct((B, 32), jnp.float32),
    )(E, mi3, subM, grM, tmb, wsub1, wgr1, W2, b2.reshape(1, 64), W3,
      b3.reshape(1, 32))


def kernel(school_idx, goal_idx, method_idx, subject_multi_hot, grade_multi_hot,
           school_emb, goal_emb, method_emb, W_sub, b_sub, W_gr, b_gr,
           W1, b1, W2, b2, W3, b3):
    si2 = school_idx.astype(jnp.int32).reshape(B // 128, 128)
    gi2 = goal_idx.astype(jnp.int32).reshape(B // 128, 128)
    mi3 = method_idx.astype(jnp.int32).reshape(B // TB, 1, TB)
    t2, tmb, wsub1, wgr1, ip2 = _prep(
        school_emb, goal_emb, method_emb, W_sub, b_sub, W_gr, b_gr, W1, b1,
        si2, gi2)
    E = _sc_gather(ip2, t2)
    return _tail(E, mi3, subject_multi_hot, grade_multi_hot,
                 tmb, wsub1, wgr1, W2, b2, W3, b3)


# speedup vs baseline: 1.1205x; 1.0050x over previous
import functools

import jax
import jax.numpy as jnp
from jax import lax
from jax.experimental import pallas as pl
from jax.experimental.pallas import tpu as pltpu
from jax.experimental.pallas import tpu_sc as plsc

B = 16384
TB = 4096
NW = 32
RPW = B // NW
NCH = RPW // 128


NS_, NG_, NM_ = 102, 22, 12
NT2 = NS_ * NG_


def _prep_body(se_ref, ge_ref, me_ref, Wsub_ref, bsub_ref, Wgr_ref, bgr_ref,
               W1_ref, b1_ref, si_ref, gi_ref,
               t2_o, tmb_o, wsub1_o, wgr1_o, ip_o):
    f32 = jnp.float32
    W1 = W1_ref[...]
    ts = jnp.dot(se_ref[...], W1[0:32, :], preferred_element_type=f32)
    tg = jnp.dot(ge_ref[...], W1[32:64, :], preferred_element_type=f32)
    bias = (b1_ref[...]
            + jnp.dot(bsub_ref[...], W1[96:128, :], preferred_element_type=f32)
            + jnp.dot(bgr_ref[...], W1[128:160, :], preferred_element_type=f32))
    tmb_o[...] = jnp.dot(me_ref[...], W1[64:96, :], preferred_element_type=f32) + bias
    t2_o[...] = (ts[:, None, :] + tg[None, :, :]).reshape(NT2, 128)
    wsub1_o[...] = jnp.dot(Wsub_ref[...], W1[96:128, :], preferred_element_type=f32)
    wgr1_o[...] = jnp.dot(Wgr_ref[...], W1[128:160, :], preferred_element_type=f32)
    ip_o[...] = si_ref[...] * NG_ + gi_ref[...]


def _prep(school_emb, goal_emb, method_emb, W_sub, b_sub, W_gr, b_gr, W1, b1,
          si2, gi2):
    return pl.pallas_call(
        _prep_body,
        out_shape=(
            jax.ShapeDtypeStruct((NT2, 128), jnp.float32),
            jax.ShapeDtypeStruct((12, 128), jnp.float32),
            jax.ShapeDtypeStruct((15, 128), jnp.float32),
            jax.ShapeDtypeStruct((12, 128), jnp.float32),
            jax.ShapeDtypeStruct((B // 128, 128), jnp.int32),
        ),
    )(school_emb, goal_emb, method_emb, W_sub, b_sub.reshape(1, 32),
      W_gr, b_gr.reshape(1, 32), W1, b1.reshape(1, 128), si2, gi2)


CH = 128
NSET = 4


def _sc_body(ip_hbm, t3_hbm, out_hbm, idx_p, bufs, gsem, wsem):
    cid = lax.axis_index("c")
    sid = lax.axis_index("s")
    wid = cid * 16 + sid
    gbase = wid * RPW
    grow = wid * NCH

    pltpu.sync_copy(ip_hbm.at[pl.ds(grow, NCH)], idx_p)

    gh = [pltpu.async_copy(t3_hbm.at[idx_p.at[p]], bufs.at[p], gsem.at[p])
          for p in range(NCH)]
    wh = []
    for q in range(NCH):
        gh[q].wait()
        wh.append(pltpu.async_copy(
            bufs.at[q], out_hbm.at[pl.ds(gbase + q * CH, CH)], wsem.at[q]))
    for h in wh:
        h.wait()


def _sc_gather(ip2, t3):
    mesh = plsc.VectorSubcoreMesh(core_axis_name="c", subcore_axis_name="s")
    k = functools.partial(
        pl.kernel,
        mesh=mesh,
        out_type=jax.ShapeDtypeStruct((B, 128), jnp.float32),
        scratch_types=[
            pltpu.VMEM((NCH, 128), jnp.int32),
            pltpu.VMEM((NSET, CH, 128), jnp.float32),
            pltpu.SemaphoreType.DMA((NSET,)),
            pltpu.SemaphoreType.DMA((NSET,)),
        ],
    )(_sc_body)
    return k(ip2, t3)


def _tail_body(e_ref, mi_ref, subM_ref, grM_ref, tmb_ref, wsub1_ref, wgr1_ref,
               W2_ref, b2_ref, W3_ref, b3_ref, out_ref):
    f32 = jnp.float32
    oh_m = (mi_ref[0, 0, :][:, None]
            == lax.broadcasted_iota(jnp.int32, (TB, 12), 1)).astype(f32)
    h1 = (e_ref[...]
          + jnp.dot(oh_m, tmb_ref[...], preferred_element_type=f32)
          + jnp.dot(subM_ref[...], wsub1_ref[...], preferred_element_type=f32)
          + jnp.dot(grM_ref[...], wgr1_ref[...], preferred_element_type=f32))
    h1 = jnp.maximum(h1, 0.0)
    h2 = jnp.maximum(jnp.dot(h1, W2_ref[...], preferred_element_type=f32) + b2_ref[...], 0.0)
    out_ref[...] = jnp.dot(h2, W3_ref[...], preferred_element_type=f32) + b3_ref[...]


def _tail(E, mi3, subM, grM, tmb, wsub1, wgr1, W2, b2, W3, b3):
    nb = B // TB

    def batch_spec(w):
        return pl.BlockSpec((TB, w), lambda i: (i, 0))

    def full_spec(shape):
        return pl.BlockSpec(shape, lambda i: (0,) * len(shape))

    return pl.pallas_call(
        _tail_body,
        grid=(nb,),
        in_specs=[
            batch_spec(128), pl.BlockSpec((1, 1, TB), lambda i: (i, 0, 0)),
            batch_spec(15), batch_spec(12),
            full_spec((12, 128)), full_spec((15, 128)), full_spec((12, 128)),
            full_spec((128, 64)), full_spec((1, 64)),
            full_spec((64, 32)), full_spec((1, 32)),
        ],
        out_specs=pl.BlockSpec((TB, 32), lambda i: (i, 0)),
        out_shape=jax.ShapeDtypeStruct((B, 32), jnp.float32),
    )(E, mi3, subM, grM, tmb, wsub1, wgr1, W2, b2.reshape(1, 64), W3,
      b3.reshape(1, 32))


def kernel(school_idx, goal_idx, method_idx, subject_multi_hot, grade_multi_hot,
           school_emb, goal_emb, method_emb, W_sub, b_sub, W_gr, b_gr,
           W1, b1, W2, b2, W3, b3):
    si2 = school_idx.astype(jnp.int32).reshape(B // 128, 128)
    gi2 = goal_idx.astype(jnp.int32).reshape(B // 128, 128)
    mi3 = method_idx.astype(jnp.int32).reshape(B // TB, 1, TB)
    t2, tmb, wsub1, wgr1, ip2 = _prep(
        school_emb, goal_emb, method_emb, W_sub, b_sub, W_gr, b_gr, W1, b1,
        si2, gi2)
    E = _sc_gather(ip2, t2)
    return _tail(E, mi3, subject_multi_hot, grade_multi_hot,
                 tmb, wsub1, wgr1, W2, b2, W3, b3)
